# baseline (speedup 1.0000x reference)
import functools

import jax
import jax.numpy as jnp
import numpy as np
from jax import lax
from jax.experimental import pallas as pl
from jax.experimental.pallas import tpu as pltpu
from jax.experimental.pallas import tpu_sc as plsc

N = 10000
E = 320000
D = 128
H = 64
C = 10
B = 64

NP = 10240
NSC = 2
NTILE = 16
TPB = NP // NTILE
CH = 80
EPW = E // (NSC * NTILE)
NCHUNK = EPW // CH

_Z = np.int32(0)

_mesh = plsc.VectorSubcoreMesh(
    core_axis_name="c", subcore_axis_name="s", num_cores=NSC,
    num_subcores=NTILE)


def _seg_body(with_cnt, p_hbm, src_hbm, dst_hbm, s_out, *rest):
  if with_cnt:
    (cnt_out, s_sh, cnt_sh, p_sh, zbuf, zbuf1, ones_v, sidx, didx, rows,
     gsem0, gsem1) = rest
  else:
    (s_sh, p_sh, zbuf, zbuf1, sidx, didx, rows, gsem0, gsem1) = rest
    cnt_out = cnt_sh = ones_v = None
  c = lax.axis_index("c")
  s = lax.axis_index("s")
  w = c * NTILE + s

  pltpu.sync_copy(p_hbm.at[pl.ds(s * TPB, TPB)], p_sh.at[pl.ds(s * TPB, TPB)])
  pltpu.sync_copy(src_hbm.at[w], sidx)
  pltpu.sync_copy(dst_hbm.at[w], didx)
  def zb(r, carry):
    for q in range(H // 16):
      zbuf[r, pl.ds(q * 16, 16)] = jnp.zeros((16,), jnp.float32)
    return carry
  lax.fori_loop(jnp.int32(0), jnp.int32(64), zb, jnp.int32(0))
  for d in range(TPB // 64):
    pltpu.sync_copy(zbuf, s_sh.at[pl.ds(s * TPB + d * 64, 64)])
  if with_cnt:
    def zb1(i, carry):
      zbuf1[pl.ds(i * 16, 16)] = jnp.zeros((16,), jnp.float32)
      return carry
    lax.fori_loop(jnp.int32(0), jnp.int32(TPB // 16), zb1, jnp.int32(0))
    for q in range(CH // 16):
      ones_v[pl.ds(q * 16, 16)] = jnp.ones((16,), jnp.float32)
    pltpu.sync_copy(zbuf1, cnt_sh.at[pl.ds(s * TPB, TPB)])
  plsc.subcore_barrier()

  sems = (gsem0, gsem1)

  def issue(k, b):
    pltpu.async_copy(p_sh.at[sidx.at[k]], rows.at[np.int32(b)], sems[b])

  def drain(b):
    pltpu.make_async_copy(p_sh.at[sidx.at[_Z]], rows.at[np.int32(b)], sems[b]).wait()

  def step(k, b):
    drain(b)
    pltpu.sync_copy(rows.at[np.int32(b)], s_sh.at[didx.at[k]], add=True)
    if with_cnt:
      pltpu.sync_copy(ones_v, cnt_sh.at[didx.at[k]], add=True)

  issue(jnp.int32(0), 0)
  issue(jnp.int32(1), 1)

  def pair(p, carry):
    k0 = p * 2
    step(k0, 0)
    issue(k0 + 2, 0)
    step(k0 + 1, 1)
    issue(k0 + 3, 1)
    return carry
  npair = (NCHUNK - 3) // 2
  lax.fori_loop(jnp.int32(0), jnp.int32(npair), pair, jnp.int32(0))
  step(jnp.int32(NCHUNK - 3), 0)
  issue(jnp.int32(NCHUNK - 1), 0)
  step(jnp.int32(NCHUNK - 2), 1)
  step(jnp.int32(NCHUNK - 1), 0)
  plsc.subcore_barrier()

  row0 = c * NP + s * TPB
  pltpu.sync_copy(s_sh.at[pl.ds(s * TPB, TPB)], s_out.at[pl.ds(row0, TPB)])
  if with_cnt:
    pltpu.sync_copy(cnt_sh.at[pl.ds(s * TPB, TPB)],
                    cnt_out.at[pl.ds(row0, TPB)])


def _seg_call(p, src, dst, with_cnt):
  out_type = [jax.ShapeDtypeStruct((NSC * NP, H), jnp.float32)]
  scratch = []
  if with_cnt:
    out_type.append(jax.ShapeDtypeStruct((NSC * NP,), jnp.float32))
  scratch.append(pltpu.VMEM_SHARED((NP, H), jnp.float32))
  if with_cnt:
    scratch.append(pltpu.VMEM_SHARED((NP,), jnp.float32))
  scratch.append(pltpu.VMEM_SHARED((NP, H), jnp.float32))
  scratch += [
      pltpu.VMEM((64, H), jnp.float32),
      pltpu.VMEM((TPB,), jnp.float32),
  ]
  if with_cnt:
    scratch.append(pltpu.VMEM((CH,), jnp.float32))
  scratch += [
      pltpu.VMEM((NCHUNK, CH), jnp.int32),
      pltpu.VMEM((NCHUNK, CH), jnp.int32),
      pltpu.VMEM((2, CH, H), jnp.float32),
      pltpu.SemaphoreType.DMA,
      pltpu.SemaphoreType.DMA,
  ]
  fn = pl.kernel(
      functools.partial(_seg_body, with_cnt),
      out_type=out_type,
      mesh=_mesh,
      scratch_types=scratch,
      compiler_params=pltpu.CompilerParams(use_tc_tiling_on_sc=False),
  )
  return fn(p, src, dst)


def _score_body(q_hbm, src_hbm, dst_hbm, g_out, g_sh, q_sh, zbuf1, sidx,
                didx, vals, gsem0, gsem1):
  c = lax.axis_index("c")
  s = lax.axis_index("s")
  w = c * NTILE + s

  pltpu.sync_copy(q_hbm.at[pl.ds(s * TPB, TPB)], q_sh.at[pl.ds(s * TPB, TPB)])
  pltpu.sync_copy(src_hbm.at[w], sidx)
  pltpu.sync_copy(dst_hbm.at[w], didx)
  def zb1(i, carry):
    zbuf1[pl.ds(i * 16, 16)] = jnp.zeros((16,), jnp.float32)
    return carry
  lax.fori_loop(jnp.int32(0), jnp.int32(TPB // 16), zb1, jnp.int32(0))
  pltpu.sync_copy(zbuf1, g_sh.at[pl.ds(s * TPB, TPB)])
  plsc.subcore_barrier()

  sems = (gsem0, gsem1)

  def issue(k, b):
    pltpu.async_copy(q_sh.at[sidx.at[k]], vals.at[np.int32(b)], sems[b])

  def step(k, b):
    pltpu.make_async_copy(q_sh.at[sidx.at[_Z]], vals.at[np.int32(b)], sems[b]).wait()
    pltpu.sync_copy(vals.at[np.int32(b)], g_sh.at[didx.at[k]], add=True)

  issue(jnp.int32(0), 0)
  issue(jnp.int32(1), 1)

  def pair(p, carry):
    k0 = p * 2
    step(k0, 0)
    issue(k0 + 2, 0)
    step(k0 + 1, 1)
    issue(k0 + 3, 1)
    return carry
  npair = (NCHUNK - 3) // 2
  lax.fori_loop(jnp.int32(0), jnp.int32(npair), pair, jnp.int32(0))
  step(jnp.int32(NCHUNK - 3), 0)
  issue(jnp.int32(NCHUNK - 1), 0)
  step(jnp.int32(NCHUNK - 2), 1)
  step(jnp.int32(NCHUNK - 1), 0)
  plsc.subcore_barrier()

  row0 = c * NP + s * TPB
  pltpu.sync_copy(g_sh.at[pl.ds(s * TPB, TPB)], g_out.at[pl.ds(row0, TPB)])


def _score_call(q, src, dst):
  fn = pl.kernel(
      _score_body,
      out_type=jax.ShapeDtypeStruct((NSC * NP,), jnp.float32),
      mesh=_mesh,
      scratch_types=[
          pltpu.VMEM_SHARED((NP,), jnp.float32),
          pltpu.VMEM_SHARED((NP,), jnp.float32),
          pltpu.VMEM((TPB,), jnp.float32),
          pltpu.VMEM((NCHUNK, CH), jnp.int32),
          pltpu.VMEM((NCHUNK, CH), jnp.int32),
          pltpu.VMEM((2, CH), jnp.float32),
          pltpu.SemaphoreType.DMA,
          pltpu.SemaphoreType.DMA,
      ],
      compiler_params=pltpu.CompilerParams(use_tc_tiling_on_sc=False),
  )
  return fn(q, src, dst)


_GRID1 = 10
_BLK1 = N // _GRID1


def _pre_body(x_ref, wl_ref, wr_ref, b_ref, p_ref, r_ref):
  xb = x_ref[...]
  p_ref[...] = jnp.dot(xb, wl_ref[...], preferred_element_type=jnp.float32)
  r_ref[...] = jnp.dot(xb, wr_ref[...],
                       preferred_element_type=jnp.float32) + b_ref[...]


def _pre_call(x, Wl, Wr, b):
  return pl.pallas_call(
      _pre_body,
      grid=(_GRID1,),
      in_specs=[
          pl.BlockSpec((_BLK1, D), lambda i: (i, _Z)),
          pl.BlockSpec((D, H), lambda i: (_Z, _Z)),
          pl.BlockSpec((D, H), lambda i: (_Z, _Z)),
          pl.BlockSpec((1, H), lambda i: (_Z, _Z)),
      ],
      out_specs=[
          pl.BlockSpec((_BLK1, H), lambda i: (i, _Z)),
          pl.BlockSpec((_BLK1, H), lambda i: (i, _Z)),
      ],
      out_shape=[
          jax.ShapeDtypeStruct((NP, H), jnp.float32),
          jax.ShapeDtypeStruct((N, H), jnp.float32),
      ],
  )(x, Wl, Wr, b)


def _mid_body(has_res, with_counts, *refs):
  it = iter(refs)
  s_ref = next(it); cnt_ref = next(it); r_ref = next(it)
  res_ref = next(it) if has_res else None
  bat_ref = next(it) if with_counts else None
  wl_ref = next(it); wr_ref = next(it); b_ref = next(it)
  x_ref = next(it); p_ref = next(it); rn_ref = next(it)
  counts_ref = next(it) if with_counts else None

  cnt = cnt_ref[0] + cnt_ref[1]
  m = (s_ref[0] + s_ref[1]) / jnp.maximum(cnt, 1.0)
  x_new = jnp.maximum(m + r_ref[...], 0.0)
  if has_res:
    x_new = x_new + res_ref[...]
  x_ref[...] = x_new
  p_ref[...] = jnp.dot(x_new, wl_ref[...], preferred_element_type=jnp.float32)
  rn_ref[...] = jnp.dot(x_new, wr_ref[...],
                        preferred_element_type=jnp.float32) + b_ref[...]
  if with_counts:
    gids = lax.broadcasted_iota(jnp.int32, (_BLK1, B), 1)
    oh = (bat_ref[...] == gids).astype(jnp.float32)
    counts_ref[...] = jnp.sum(oh, axis=0, keepdims=True)[None]


def _mid_call(s_pair, cnt_pair, r, res, bat, Wl, Wr, b):
  has_res = res is not None
  with_counts = bat is not None
  hn = Wl.shape[1]
  in_specs = [
      pl.BlockSpec((NSC, _BLK1, H), lambda i: (_Z, i, _Z)),
      pl.BlockSpec((NSC, _BLK1, 1), lambda i: (_Z, i, _Z)),
      pl.BlockSpec((_BLK1, H), lambda i: (i, _Z)),
  ]
  args = [s_pair, cnt_pair, r]
  if has_res:
    in_specs.append(pl.BlockSpec((_BLK1, H), lambda i: (i, _Z)))
    args.append(res)
  if with_counts:
    in_specs.append(pl.BlockSpec((_BLK1, 1), lambda i: (i, _Z)))
    args.append(bat)
  in_specs += [
      pl.BlockSpec((H, hn), lambda i: (_Z, _Z)),
      pl.BlockSpec((H, hn), lambda i: (_Z, _Z)),
      pl.BlockSpec((1, hn), lambda i: (_Z, _Z)),
  ]
  args += [Wl, Wr, b]
  out_specs = [
      pl.BlockSpec((_BLK1, H), lambda i: (i, _Z)),
      pl.BlockSpec((_BLK1, hn), lambda i: (i, _Z)),
      pl.BlockSpec((_BLK1, hn), lambda i: (i, _Z)),
  ]
  out_shape = [
      jax.ShapeDtypeStruct((N, H), jnp.float32),
      jax.ShapeDtypeStruct((NP, hn), jnp.float32),
      jax.ShapeDtypeStruct((N, hn), jnp.float32),
  ]
  if with_counts:
    out_specs.append(pl.BlockSpec((1, 1, B), lambda i: (i, _Z, _Z)))
    out_shape.append(jax.ShapeDtypeStruct((_GRID1, 1, B), jnp.float32))
  return pl.pallas_call(
      functools.partial(_mid_body, has_res, with_counts),
      grid=(_GRID1,),
      in_specs=in_specs,
      out_specs=out_specs,
      out_shape=out_shape,
  )(*args)


_GRIDF = 25
_BLKF = N // _GRIDF
_JW = 512
_NJ = NP // _JW


def _final_body(x3_ref, saggc_ref, qpc_ref, batc_ref, saggr_ref, qpr_ref,
                batr_ref, counts_ref, wlin_ref, blin_ref, out_ref,
                sums_ref, cntk_ref):
  pid = pl.program_id(0)

  score_c = saggc_ref[0] + saggc_ref[1] + qpc_ref[...]
  b_c = batc_ref[...]
  i_c = lax.broadcasted_iota(jnp.int32, (_BLKF, 1), 0) + pid * _BLKF

  bimin = jnp.min(b_c)
  bimax = jnp.max(b_c)

  def jstep(jt, acc):
    sr = saggr_ref[0, jt] + saggr_ref[1, jt] + qpr_ref[jt]
    br = batr_ref[jt]
    ir = lax.broadcasted_iota(jnp.int32, (1, _JW), 1) + jt * _JW
    overlap = (jnp.min(br) <= bimax) & (jnp.max(br) >= bimin)

    def hit(a):
      gt = (sr > score_c) | ((sr == score_c) & (ir < i_c))
      same = br == b_c
      return a + jnp.sum((gt & same).astype(jnp.float32), axis=1,
                         keepdims=True)
    return lax.cond(overlap, hit, lambda a: a, acc)

  rank = lax.fori_loop(jnp.int32(0), jnp.int32(_NJ), jstep,
                       jnp.zeros((_BLKF, 1), jnp.float32))

  counts = jnp.sum(counts_ref[...], axis=0)
  k_per = jnp.floor((counts + 1.0) * 0.5)
  gids = lax.broadcasted_iota(jnp.int32, (_BLKF, B), 1)
  oh = (b_c == gids).astype(jnp.float32)
  k_node = jnp.sum(oh * k_per, axis=1, keepdims=True)

  keep = (rank < k_node).astype(jnp.float32)
  gate = jnp.tanh(score_c) * keep
  gated = x3_ref[...] * gate

  part_sums = lax.dot_general(oh, gated, (((0,), (0,)), ((), ())),
                              preferred_element_type=jnp.float32)
  part_cnt = jnp.sum(oh * keep, axis=0, keepdims=True)

  @pl.when(pid == 0)
  def _init():
    sums_ref[...] = part_sums
    cntk_ref[...] = part_cnt

  @pl.when(pid > 0)
  def _acc():
    sums_ref[...] += part_sums
    cntk_ref[...] += part_cnt

  @pl.when(pid == _GRIDF - 1)
  def _fin():
    denom = jnp.maximum(cntk_ref[...], 1.0)
    pooled = sums_ref[...] / denom.reshape(B, 1)
    logits = jnp.dot(pooled, wlin_ref[...],
                     preferred_element_type=jnp.float32) + blin_ref[...]
    mx = jnp.max(logits, axis=1, keepdims=True)
    lse = mx + jnp.log(jnp.sum(jnp.exp(logits - mx), axis=1, keepdims=True))
    out_ref[...] = logits - lse


def _final_call(x3, sagg_col, qpob_col, bat_col, sagg_row, qpob_row, bat_row,
                counts, Wlin, blin):
  return pl.pallas_call(
      _final_body,
      grid=(_GRIDF,),
      in_specs=[
          pl.BlockSpec((_BLKF, H), lambda i: (i, _Z)),
          pl.BlockSpec((NSC, _BLKF, 1), lambda i: (_Z, i, _Z)),
          pl.BlockSpec((_BLKF, 1), lambda i: (i, _Z)),
          pl.BlockSpec((_BLKF, 1), lambda i: (i, _Z)),
          pl.BlockSpec((NSC, _NJ, 1, _JW), lambda i: (_Z, _Z, _Z, _Z)),
          pl.BlockSpec((_NJ, 1, _JW), lambda i: (_Z, _Z, _Z)),
          pl.BlockSpec((_NJ, 1, _JW), lambda i: (_Z, _Z, _Z)),
          pl.BlockSpec((_GRID1, 1, B), lambda i: (_Z, _Z, _Z)),
          pl.BlockSpec((H, C), lambda i: (_Z, _Z)),
          pl.BlockSpec((1, C), lambda i: (_Z, _Z)),
      ],
      out_specs=pl.BlockSpec((B, C), lambda i: (_Z, _Z)),
      out_shape=jax.ShapeDtypeStruct((B, C), jnp.float32),
      scratch_shapes=[
          pltpu.VMEM((B, H), jnp.float32),
          pltpu.VMEM((1, B), jnp.float32),
      ],
  )(x3, sagg_col, qpob_col, bat_col, sagg_row, qpob_row, bat_row, counts,
    Wlin, blin)


def kernel(x, edge_index, batch, W1l, b1, W1r, W2l, b2, W2r, W3l, b3, W3r,
           Wpr, bpr, Wpo, Wlin, blin):
  x = x.astype(jnp.float32)
  src = edge_index[0].astype(jnp.int32).reshape(NSC * NTILE, NCHUNK, CH)
  dst = edge_index[1].astype(jnp.int32).reshape(NSC * NTILE, NCHUNK, CH)
  bat = batch.astype(jnp.int32)
  bat_col = bat.reshape(N, 1)
  bat_row = jnp.pad(bat, (0, NP - N), constant_values=-1).reshape(_NJ, 1, _JW)

  p1, r1 = _pre_call(x, W1l, W1r, b1.reshape(1, H))
  s1_flat, cnt_flat = _seg_call(p1, src, dst, with_cnt=True)
  s1 = s1_flat.reshape(NSC, NP, H)
  cnt = cnt_flat.reshape(NSC, NP, 1)

  x1, p2, r2 = _mid_call(s1, cnt, r1, None, None, W2l, W2r, b2.reshape(1, H))
  s2 = _seg_call(p2, src, dst, with_cnt=False)[0].reshape(NSC, NP, H)

  x2, p3, r3 = _mid_call(s2, cnt, r2, x1, None, W3l, W3r, b3.reshape(1, H))
  s3 = _seg_call(p3, src, dst, with_cnt=False)[0].reshape(NSC, NP, H)

  x3, qpr, qpob, counts = _mid_call(
      s3, cnt, r3, x2, bat_col, Wpr, Wpo, bpr.reshape(1, 1))

  sagg_flat = _score_call(qpr.reshape(NP), src, dst)
  sagg_col = sagg_flat.reshape(NSC, NP, 1)
  sagg_row = sagg_flat.reshape(NSC, _NJ, 1, _JW)
  qpob_row = jnp.pad(qpob.reshape(N), (0, NP - N)).reshape(_NJ, 1, _JW)

  return _final_call(x3, sagg_col, qpob.reshape(N, 1), bat_col, sagg_row,
                     qpob_row, bat_row, counts, Wlin, blin.reshape(1, C))


# speedup vs baseline: 1.0850x; 1.0850x over previous
import functools

import jax
import jax.numpy as jnp
import numpy as np
from jax import lax
from jax.experimental import pallas as pl
from jax.experimental.pallas import tpu as pltpu
from jax.experimental.pallas import tpu_sc as plsc

N = 10000
E = 320000
D = 128
H = 64
C = 10
B = 64

NP = 10240
NSC = 2
NTILE = 16
TPB = NP // NTILE
CH = 80
EPW = E // (NSC * NTILE)
NCHUNK = EPW // CH

_Z = np.int32(0)

_mesh = plsc.VectorSubcoreMesh(
    core_axis_name="c", subcore_axis_name="s", num_cores=NSC,
    num_subcores=NTILE)


def _seg_body(with_cnt, p_hbm, src_hbm, dst_hbm, s_out, *rest):
  if with_cnt:
    (cnt_out, s_sh, cnt_sh, p_sh, zbuf, zbuf1, ones_v, sidx, didx, rows,
     gsem0, gsem1) = rest
  else:
    (s_sh, p_sh, zbuf, zbuf1, sidx, didx, rows, gsem0, gsem1) = rest
    cnt_out = cnt_sh = ones_v = None
  c = lax.axis_index("c")
  s = lax.axis_index("s")
  w = c * NTILE + s

  pltpu.sync_copy(p_hbm.at[pl.ds(s * TPB, TPB)], p_sh.at[pl.ds(s * TPB, TPB)])
  pltpu.sync_copy(src_hbm.at[w], sidx)
  pltpu.sync_copy(dst_hbm.at[w], didx)
  def zb(r, carry):
    for q in range(H // 16):
      zbuf[r, pl.ds(q * 16, 16)] = jnp.zeros((16,), jnp.float32)
    return carry
  lax.fori_loop(jnp.int32(0), jnp.int32(64), zb, jnp.int32(0))
  for d in range(TPB // 64):
    pltpu.sync_copy(zbuf, s_sh.at[pl.ds(s * TPB + d * 64, 64)])
  if with_cnt:
    def zb1(i, carry):
      zbuf1[pl.ds(i * 16, 16)] = jnp.zeros((16,), jnp.float32)
      return carry
    lax.fori_loop(jnp.int32(0), jnp.int32(TPB // 16), zb1, jnp.int32(0))
    for q in range(CH // 16):
      ones_v[pl.ds(q * 16, 16)] = jnp.ones((16,), jnp.float32)
    pltpu.sync_copy(zbuf1, cnt_sh.at[pl.ds(s * TPB, TPB)])
  plsc.subcore_barrier()

  sems = (gsem0, gsem1)

  def issue(k, b):
    pltpu.async_copy(p_sh.at[sidx.at[k]], rows.at[np.int32(b)], sems[b])

  def drain(b):
    pltpu.make_async_copy(p_sh.at[sidx.at[_Z]], rows.at[np.int32(b)], sems[b]).wait()

  def step(k, b):
    drain(b)
    pltpu.sync_copy(rows.at[np.int32(b)], s_sh.at[didx.at[k]], add=True)
    if with_cnt:
      pltpu.sync_copy(ones_v, cnt_sh.at[didx.at[k]], add=True)

  issue(jnp.int32(0), 0)
  issue(jnp.int32(1), 1)

  def pair(p, carry):
    k0 = p * 2
    step(k0, 0)
    issue(k0 + 2, 0)
    step(k0 + 1, 1)
    issue(k0 + 3, 1)
    return carry
  npair = (NCHUNK - 3) // 2
  lax.fori_loop(jnp.int32(0), jnp.int32(npair), pair, jnp.int32(0))
  step(jnp.int32(NCHUNK - 3), 0)
  issue(jnp.int32(NCHUNK - 1), 0)
  step(jnp.int32(NCHUNK - 2), 1)
  step(jnp.int32(NCHUNK - 1), 0)
  plsc.subcore_barrier()

  row0 = c * NP + s * TPB
  pltpu.sync_copy(s_sh.at[pl.ds(s * TPB, TPB)], s_out.at[pl.ds(row0, TPB)])
  if with_cnt:
    pltpu.sync_copy(cnt_sh.at[pl.ds(s * TPB, TPB)],
                    cnt_out.at[pl.ds(row0, TPB)])


def _seg_call(p, src, dst, with_cnt):
  out_type = [jax.ShapeDtypeStruct((NSC * NP, H), jnp.float32)]
  scratch = []
  if with_cnt:
    out_type.append(jax.ShapeDtypeStruct((NSC * NP,), jnp.float32))
  scratch.append(pltpu.VMEM_SHARED((NP, H), jnp.float32))
  if with_cnt:
    scratch.append(pltpu.VMEM_SHARED((NP,), jnp.float32))
  scratch.append(pltpu.VMEM_SHARED((NP, H), jnp.float32))
  scratch += [
      pltpu.VMEM((64, H), jnp.float32),
      pltpu.VMEM((TPB,), jnp.float32),
  ]
  if with_cnt:
    scratch.append(pltpu.VMEM((CH,), jnp.float32))
  scratch += [
      pltpu.VMEM((NCHUNK, CH), jnp.int32),
      pltpu.VMEM((NCHUNK, CH), jnp.int32),
      pltpu.VMEM((2, CH, H), jnp.float32),
      pltpu.SemaphoreType.DMA,
      pltpu.SemaphoreType.DMA,
  ]
  fn = pl.kernel(
      functools.partial(_seg_body, with_cnt),
      out_type=out_type,
      mesh=_mesh,
      scratch_types=scratch,
      compiler_params=pltpu.CompilerParams(use_tc_tiling_on_sc=False),
  )
  return fn(p, src, dst)


def _score_body(q_hbm, src_hbm, dst_hbm, g_out, g_sh, q_sh, zbuf1, sidx,
                didx, vals, gsem0, gsem1):
  c = lax.axis_index("c")
  s = lax.axis_index("s")
  w = c * NTILE + s

  pltpu.sync_copy(q_hbm.at[pl.ds(s * TPB, TPB)], q_sh.at[pl.ds(s * TPB, TPB)])
  pltpu.sync_copy(src_hbm.at[w], sidx)
  pltpu.sync_copy(dst_hbm.at[w], didx)
  def zb1(i, carry):
    zbuf1[pl.ds(i * 16, 16)] = jnp.zeros((16,), jnp.float32)
    return carry
  lax.fori_loop(jnp.int32(0), jnp.int32(TPB // 16), zb1, jnp.int32(0))
  pltpu.sync_copy(zbuf1, g_sh.at[pl.ds(s * TPB, TPB)])
  plsc.subcore_barrier()

  sems = (gsem0, gsem1)

  def issue(k, b):
    pltpu.async_copy(q_sh.at[sidx.at[k]], vals.at[np.int32(b)], sems[b])

  def step(k, b):
    pltpu.make_async_copy(q_sh.at[sidx.at[_Z]], vals.at[np.int32(b)], sems[b]).wait()
    pltpu.sync_copy(vals.at[np.int32(b)], g_sh.at[didx.at[k]], add=True)

  issue(jnp.int32(0), 0)
  issue(jnp.int32(1), 1)

  def pair(p, carry):
    k0 = p * 2
    step(k0, 0)
    issue(k0 + 2, 0)
    step(k0 + 1, 1)
    issue(k0 + 3, 1)
    return carry
  npair = (NCHUNK - 3) // 2
  lax.fori_loop(jnp.int32(0), jnp.int32(npair), pair, jnp.int32(0))
  step(jnp.int32(NCHUNK - 3), 0)
  issue(jnp.int32(NCHUNK - 1), 0)
  step(jnp.int32(NCHUNK - 2), 1)
  step(jnp.int32(NCHUNK - 1), 0)
  plsc.subcore_barrier()

  row0 = c * NP + s * TPB
  pltpu.sync_copy(g_sh.at[pl.ds(s * TPB, TPB)], g_out.at[pl.ds(row0, TPB)])


def _score_call(q, src, dst):
  fn = pl.kernel(
      _score_body,
      out_type=jax.ShapeDtypeStruct((NSC * NP,), jnp.float32),
      mesh=_mesh,
      scratch_types=[
          pltpu.VMEM_SHARED((NP,), jnp.float32),
          pltpu.VMEM_SHARED((NP,), jnp.float32),
          pltpu.VMEM((TPB,), jnp.float32),
          pltpu.VMEM((NCHUNK, CH), jnp.int32),
          pltpu.VMEM((NCHUNK, CH), jnp.int32),
          pltpu.VMEM((2, CH), jnp.float32),
          pltpu.SemaphoreType.DMA,
          pltpu.SemaphoreType.DMA,
      ],
      compiler_params=pltpu.CompilerParams(use_tc_tiling_on_sc=False),
  )
  return fn(q, src, dst)


_GRID1 = 10
_BLK1 = N // _GRID1


def _pre_body(x_ref, wl_ref, wr_ref, b_ref, p_ref, r_ref):
  xb = x_ref[...]
  p_ref[...] = jnp.dot(xb, wl_ref[...], preferred_element_type=jnp.float32)
  r_ref[...] = jnp.dot(xb, wr_ref[...],
                       preferred_element_type=jnp.float32) + b_ref[...]


def _pre_call(x, Wl, Wr, b):
  return pl.pallas_call(
      _pre_body,
      grid=(_GRID1,),
      in_specs=[
          pl.BlockSpec((_BLK1, D), lambda i: (i, _Z)),
          pl.BlockSpec((D, H), lambda i: (_Z, _Z)),
          pl.BlockSpec((D, H), lambda i: (_Z, _Z)),
          pl.BlockSpec((1, H), lambda i: (_Z, _Z)),
      ],
      out_specs=[
          pl.BlockSpec((_BLK1, H), lambda i: (i, _Z)),
          pl.BlockSpec((_BLK1, H), lambda i: (i, _Z)),
      ],
      out_shape=[
          jax.ShapeDtypeStruct((NP, H), jnp.float32),
          jax.ShapeDtypeStruct((N, H), jnp.float32),
      ],
  )(x, Wl, Wr, b)


def _mid_body(has_res, with_counts, *refs):
  it = iter(refs)
  s_ref = next(it); cnt_ref = next(it); r_ref = next(it)
  res_ref = next(it) if has_res else None
  bat_ref = next(it) if with_counts else None
  wl_ref = next(it); wr_ref = next(it); b_ref = next(it)
  x_ref = next(it); p_ref = next(it); rn_ref = next(it)
  counts_ref = next(it) if with_counts else None

  cnt = cnt_ref[0] + cnt_ref[1]
  m = (s_ref[0] + s_ref[1]) / jnp.maximum(cnt, 1.0)
  x_new = jnp.maximum(m + r_ref[...], 0.0)
  if has_res:
    x_new = x_new + res_ref[...]
  x_ref[...] = x_new
  p_ref[...] = jnp.dot(x_new, wl_ref[...], preferred_element_type=jnp.float32)
  rn_ref[...] = jnp.dot(x_new, wr_ref[...],
                        preferred_element_type=jnp.float32) + b_ref[...]
  if with_counts:
    gids = lax.broadcasted_iota(jnp.int32, (_BLK1, B), 1)
    oh = (bat_ref[...] == gids).astype(jnp.float32)
    counts_ref[...] = jnp.sum(oh, axis=0, keepdims=True)[None]


def _mid_call(s_pair, cnt_pair, r, res, bat, Wl, Wr, b):
  has_res = res is not None
  with_counts = bat is not None
  hn = Wl.shape[1]
  in_specs = [
      pl.BlockSpec((NSC, _BLK1, H), lambda i: (_Z, i, _Z)),
      pl.BlockSpec((NSC, _BLK1, 1), lambda i: (_Z, i, _Z)),
      pl.BlockSpec((_BLK1, H), lambda i: (i, _Z)),
  ]
  args = [s_pair, cnt_pair, r]
  if has_res:
    in_specs.append(pl.BlockSpec((_BLK1, H), lambda i: (i, _Z)))
    args.append(res)
  if with_counts:
    in_specs.append(pl.BlockSpec((_BLK1, 1), lambda i: (i, _Z)))
    args.append(bat)
  in_specs += [
      pl.BlockSpec((H, hn), lambda i: (_Z, _Z)),
      pl.BlockSpec((H, hn), lambda i: (_Z, _Z)),
      pl.BlockSpec((1, hn), lambda i: (_Z, _Z)),
  ]
  args += [Wl, Wr, b]
  out_specs = [
      pl.BlockSpec((_BLK1, H), lambda i: (i, _Z)),
      pl.BlockSpec((_BLK1, hn), lambda i: (i, _Z)),
      pl.BlockSpec((_BLK1, hn), lambda i: (i, _Z)),
  ]
  out_shape = [
      jax.ShapeDtypeStruct((N, H), jnp.float32),
      jax.ShapeDtypeStruct((NP, hn), jnp.float32),
      jax.ShapeDtypeStruct((N, hn), jnp.float32),
  ]
  if with_counts:
    out_specs.append(pl.BlockSpec((1, 1, B), lambda i: (i, _Z, _Z)))
    out_shape.append(jax.ShapeDtypeStruct((_GRID1, 1, B), jnp.float32))
  return pl.pallas_call(
      functools.partial(_mid_body, has_res, with_counts),
      grid=(_GRID1,),
      in_specs=in_specs,
      out_specs=out_specs,
      out_shape=out_shape,
  )(*args)


_GRIDF = 25
_BLKF = N // _GRIDF
_JW = 1024
_NJ = NP // _JW


def _final_body(x3_ref, saggc_ref, qpc_ref, batc_ref, saggr_ref, qpr_ref,
                batr_ref, counts_ref, wlin_ref, blin_ref, out_ref,
                sums_ref, cntk_ref):
  pid = pl.program_id(0)

  score_c = saggc_ref[0] + saggc_ref[1] + qpc_ref[...]
  b_c = batc_ref[...]
  i_c = lax.broadcasted_iota(jnp.int32, (_BLKF, 1), 0) + pid * _BLKF

  bimin = jnp.min(b_c)
  bimax = jnp.max(b_c)

  def jstep(jt, acc):
    sr = saggr_ref[0, jt] + saggr_ref[1, jt] + qpr_ref[jt]
    br = batr_ref[jt]
    ir = lax.broadcasted_iota(jnp.int32, (1, _JW), 1) + jt * _JW
    overlap = (jnp.min(br) <= bimax) & (jnp.max(br) >= bimin)

    def hit(a):
      gt = (sr > score_c) | ((sr == score_c) & (ir < i_c))
      same = br == b_c
      return a + jnp.sum((gt & same).astype(jnp.float32), axis=1,
                         keepdims=True)
    return lax.cond(overlap, hit, lambda a: a, acc)

  rank = lax.fori_loop(jnp.int32(0), jnp.int32(_NJ), jstep,
                       jnp.zeros((_BLKF, 1), jnp.float32))

  counts = jnp.sum(counts_ref[...], axis=0)
  k_per = jnp.floor((counts + 1.0) * 0.5)
  gids = lax.broadcasted_iota(jnp.int32, (_BLKF, B), 1)
  oh = (b_c == gids).astype(jnp.float32)
  k_node = jnp.sum(oh * k_per, axis=1, keepdims=True)

  keep = (rank < k_node).astype(jnp.float32)
  gate = jnp.tanh(score_c) * keep
  gated = x3_ref[...] * gate

  part_sums = lax.dot_general(oh, gated, (((0,), (0,)), ((), ())),
                              preferred_element_type=jnp.float32)
  part_cnt = jnp.sum(oh * keep, axis=0, keepdims=True)

  @pl.when(pid == 0)
  def _init():
    sums_ref[...] = part_sums
    cntk_ref[...] = part_cnt

  @pl.when(pid > 0)
  def _acc():
    sums_ref[...] += part_sums
    cntk_ref[...] += part_cnt

  @pl.when(pid == _GRIDF - 1)
  def _fin():
    denom = jnp.maximum(cntk_ref[...], 1.0)
    pooled = sums_ref[...] / denom.reshape(B, 1)
    logits = jnp.dot(pooled, wlin_ref[...],
                     preferred_element_type=jnp.float32) + blin_ref[...]
    mx = jnp.max(logits, axis=1, keepdims=True)
    lse = mx + jnp.log(jnp.sum(jnp.exp(logits - mx), axis=1, keepdims=True))
    out_ref[...] = logits - lse


def _final_call(x3, sagg_col, qpob_col, bat_col, sagg_row, qpob_row, bat_row,
                counts, Wlin, blin):
  return pl.pallas_call(
      _final_body,
      grid=(_GRIDF,),
      in_specs=[
          pl.BlockSpec((_BLKF, H), lambda i: (i, _Z)),
          pl.BlockSpec((NSC, _BLKF, 1), lambda i: (_Z, i, _Z)),
          pl.BlockSpec((_BLKF, 1), lambda i: (i, _Z)),
          pl.BlockSpec((_BLKF, 1), lambda i: (i, _Z)),
          pl.BlockSpec((NSC, _NJ, 1, _JW), lambda i: (_Z, _Z, _Z, _Z)),
          pl.BlockSpec((_NJ, 1, _JW), lambda i: (_Z, _Z, _Z)),
          pl.BlockSpec((_NJ, 1, _JW), lambda i: (_Z, _Z, _Z)),
          pl.BlockSpec((_GRID1, 1, B), lambda i: (_Z, _Z, _Z)),
          pl.BlockSpec((H, C), lambda i: (_Z, _Z)),
          pl.BlockSpec((1, C), lambda i: (_Z, _Z)),
      ],
      out_specs=pl.BlockSpec((B, C), lambda i: (_Z, _Z)),
      out_shape=jax.ShapeDtypeStruct((B, C), jnp.float32),
      scratch_shapes=[
          pltpu.VMEM((B, H), jnp.float32),
          pltpu.VMEM((1, B), jnp.float32),
      ],
  )(x3, sagg_col, qpob_col, bat_col, sagg_row, qpob_row, bat_row, counts,
    Wlin, blin)


def kernel(x, edge_index, batch, W1l, b1, W1r, W2l, b2, W2r, W3l, b3, W3r,
           Wpr, bpr, Wpo, Wlin, blin):
  x = x.astype(jnp.float32)
  src = edge_index[0].astype(jnp.int32).reshape(NSC * NTILE, NCHUNK, CH)
  dst = edge_index[1].astype(jnp.int32).reshape(NSC * NTILE, NCHUNK, CH)
  bat = batch.astype(jnp.int32)
  bat_col = bat.reshape(N, 1)
  bat_row = jnp.pad(bat, (0, NP - N), constant_values=-1).reshape(_NJ, 1, _JW)

  p1, r1 = _pre_call(x, W1l, W1r, b1.reshape(1, H))
  s1_flat, cnt_flat = _seg_call(p1, src, dst, with_cnt=True)
  s1 = s1_flat.reshape(NSC, NP, H)
  cnt = cnt_flat.reshape(NSC, NP, 1)

  x1, p2, r2 = _mid_call(s1, cnt, r1, None, None, W2l, W2r, b2.reshape(1, H))
  s2 = _seg_call(p2, src, dst, with_cnt=False)[0].reshape(NSC, NP, H)

  x2, p3, r3 = _mid_call(s2, cnt, r2, x1, None, W3l, W3r, b3.reshape(1, H))
  s3 = _seg_call(p3, src, dst, with_cnt=False)[0].reshape(NSC, NP, H)

  x3, qpr, qpob, counts = _mid_call(
      s3, cnt, r3, x2, bat_col, Wpr, Wpo, bpr.reshape(1, 1))

  sagg_flat = _score_call(qpr.reshape(NP), src, dst)
  sagg_col = sagg_flat.reshape(NSC, NP, 1)
  sagg_row = sagg_flat.reshape(NSC, _NJ, 1, _JW)
  qpob_row = jnp.pad(qpob.reshape(N), (0, NP - N)).reshape(_NJ, 1, _JW)

  return _final_call(x3, sagg_col, qpob.reshape(N, 1), bat_col, sagg_row,
                     qpob_row, bat_row, counts, Wlin, blin.reshape(1, C))


# speedup vs baseline: 1.0927x; 1.0071x over previous
import functools

import jax
import jax.numpy as jnp
import numpy as np
from jax import lax
from jax.experimental import pallas as pl
from jax.experimental.pallas import tpu as pltpu
from jax.experimental.pallas import tpu_sc as plsc

N = 10000
E = 320000
D = 128
H = 64
C = 10
B = 64

NP = 10240
NSC = 2
NTILE = 16
TPB = NP // NTILE
CH = 80
EPW = E // (NSC * NTILE)
NCHUNK = EPW // CH

_Z = np.int32(0)

_mesh = plsc.VectorSubcoreMesh(
    core_axis_name="c", subcore_axis_name="s", num_cores=NSC,
    num_subcores=NTILE)


def _seg_body(with_cnt, p_hbm, src_hbm, dst_hbm, s_out, *rest):
  if with_cnt:
    (cnt_out, s_sh, cnt_sh, p_sh, zbuf, zbuf1, ones_v, sidx, didx, rows,
     gsem0, gsem1, gsem2, gsem3) = rest
  else:
    (s_sh, p_sh, zbuf, zbuf1, sidx, didx, rows, gsem0, gsem1, gsem2,
     gsem3) = rest
    cnt_out = cnt_sh = ones_v = None
  c = lax.axis_index("c")
  s = lax.axis_index("s")
  w = c * NTILE + s

  pltpu.sync_copy(p_hbm.at[pl.ds(s * TPB, TPB)], p_sh.at[pl.ds(s * TPB, TPB)])
  pltpu.sync_copy(src_hbm.at[w], sidx)
  pltpu.sync_copy(dst_hbm.at[w], didx)
  def zb(r, carry):
    for q in range(H // 16):
      zbuf[r, pl.ds(q * 16, 16)] = jnp.zeros((16,), jnp.float32)
    return carry
  lax.fori_loop(jnp.int32(0), jnp.int32(64), zb, jnp.int32(0))
  for d in range(TPB // 64):
    pltpu.sync_copy(zbuf, s_sh.at[pl.ds(s * TPB + d * 64, 64)])
  if with_cnt:
    def zb1(i, carry):
      zbuf1[pl.ds(i * 16, 16)] = jnp.zeros((16,), jnp.float32)
      return carry
    lax.fori_loop(jnp.int32(0), jnp.int32(TPB // 16), zb1, jnp.int32(0))
    for q in range(CH // 16):
      ones_v[pl.ds(q * 16, 16)] = jnp.ones((16,), jnp.float32)
    pltpu.sync_copy(zbuf1, cnt_sh.at[pl.ds(s * TPB, TPB)])
  plsc.subcore_barrier()

  sems = (gsem0, gsem1, gsem2, gsem3)

  def issue(k, b):
    pltpu.async_copy(p_sh.at[sidx.at[k]], rows.at[np.int32(b)], sems[b])

  def step(k, b):
    pltpu.make_async_copy(p_sh.at[sidx.at[_Z]], rows.at[np.int32(b)],
                          sems[b]).wait()
    pltpu.sync_copy(rows.at[np.int32(b)], s_sh.at[didx.at[k]], add=True)
    if with_cnt:
      pltpu.sync_copy(ones_v, cnt_sh.at[didx.at[k]], add=True)

  for b in range(4):
    issue(jnp.int32(b), b)

  def quad(p, carry):
    k0 = p * 4
    for b in range(4):
      step(k0 + b, b)
      issue(k0 + b + 4, b)
    return carry
  nquad = (NCHUNK - 5) // 4
  lax.fori_loop(jnp.int32(0), jnp.int32(nquad), quad, jnp.int32(0))
  step(jnp.int32(NCHUNK - 5), 0)
  issue(jnp.int32(NCHUNK - 1), 0)
  step(jnp.int32(NCHUNK - 4), 1)
  step(jnp.int32(NCHUNK - 3), 2)
  step(jnp.int32(NCHUNK - 2), 3)
  step(jnp.int32(NCHUNK - 1), 0)
  plsc.subcore_barrier()

  row0 = c * NP + s * TPB
  pltpu.sync_copy(s_sh.at[pl.ds(s * TPB, TPB)], s_out.at[pl.ds(row0, TPB)])
  if with_cnt:
    pltpu.sync_copy(cnt_sh.at[pl.ds(s * TPB, TPB)],
                    cnt_out.at[pl.ds(row0, TPB)])


def _seg_call(p, src, dst, with_cnt):
  out_type = [jax.ShapeDtypeStruct((NSC * NP, H), jnp.float32)]
  scratch = []
  if with_cnt:
    out_type.append(jax.ShapeDtypeStruct((NSC * NP,), jnp.float32))
  scratch.append(pltpu.VMEM_SHARED((NP, H), jnp.float32))
  if with_cnt:
    scratch.append(pltpu.VMEM_SHARED((NP,), jnp.float32))
  scratch.append(pltpu.VMEM_SHARED((NP, H), jnp.float32))
  scratch += [
      pltpu.VMEM((64, H), jnp.float32),
      pltpu.VMEM((TPB,), jnp.float32),
  ]
  if with_cnt:
    scratch.append(pltpu.VMEM((CH,), jnp.float32))
  scratch += [
      pltpu.VMEM((NCHUNK, CH), jnp.int32),
      pltpu.VMEM((NCHUNK, CH), jnp.int32),
      pltpu.VMEM((4, CH, H), jnp.float32),
      pltpu.SemaphoreType.DMA,
      pltpu.SemaphoreType.DMA,
      pltpu.SemaphoreType.DMA,
      pltpu.SemaphoreType.DMA,
  ]
  fn = pl.kernel(
      functools.partial(_seg_body, with_cnt),
      out_type=out_type,
      mesh=_mesh,
      scratch_types=scratch,
      compiler_params=pltpu.CompilerParams(use_tc_tiling_on_sc=False),
  )
  return fn(p, src, dst)


def _score_body(q_hbm, src_hbm, dst_hbm, g_out, g_sh, q_sh, zbuf1, sidx,
                didx, vals, gsem0, gsem1, gsem2, gsem3):
  c = lax.axis_index("c")
  s = lax.axis_index("s")
  w = c * NTILE + s

  pltpu.sync_copy(q_hbm.at[pl.ds(s * TPB, TPB)], q_sh.at[pl.ds(s * TPB, TPB)])
  pltpu.sync_copy(src_hbm.at[w], sidx)
  pltpu.sync_copy(dst_hbm.at[w], didx)
  def zb1(i, carry):
    zbuf1[pl.ds(i * 16, 16)] = jnp.zeros((16,), jnp.float32)
    return carry
  lax.fori_loop(jnp.int32(0), jnp.int32(TPB // 16), zb1, jnp.int32(0))
  pltpu.sync_copy(zbuf1, g_sh.at[pl.ds(s * TPB, TPB)])
  plsc.subcore_barrier()

  sems = (gsem0, gsem1, gsem2, gsem3)

  def issue(k, b):
    pltpu.async_copy(q_sh.at[sidx.at[k]], vals.at[np.int32(b)], sems[b])

  def step(k, b):
    pltpu.make_async_copy(q_sh.at[sidx.at[_Z]], vals.at[np.int32(b)],
                          sems[b]).wait()
    pltpu.sync_copy(vals.at[np.int32(b)], g_sh.at[didx.at[k]], add=True)

  for b in range(4):
    issue(jnp.int32(b), b)

  def quad(p, carry):
    k0 = p * 4
    for b in range(4):
      step(k0 + b, b)
      issue(k0 + b + 4, b)
    return carry
  nquad = (NCHUNK - 5) // 4
  lax.fori_loop(jnp.int32(0), jnp.int32(nquad), quad, jnp.int32(0))
  step(jnp.int32(NCHUNK - 5), 0)
  issue(jnp.int32(NCHUNK - 1), 0)
  step(jnp.int32(NCHUNK - 4), 1)
  step(jnp.int32(NCHUNK - 3), 2)
  step(jnp.int32(NCHUNK - 2), 3)
  step(jnp.int32(NCHUNK - 1), 0)
  plsc.subcore_barrier()

  row0 = c * NP + s * TPB
  pltpu.sync_copy(g_sh.at[pl.ds(s * TPB, TPB)], g_out.at[pl.ds(row0, TPB)])


def _score_call(q, src, dst):
  fn = pl.kernel(
      _score_body,
      out_type=jax.ShapeDtypeStruct((NSC * NP,), jnp.float32),
      mesh=_mesh,
      scratch_types=[
          pltpu.VMEM_SHARED((NP,), jnp.float32),
          pltpu.VMEM_SHARED((NP,), jnp.float32),
          pltpu.VMEM((TPB,), jnp.float32),
          pltpu.VMEM((NCHUNK, CH), jnp.int32),
          pltpu.VMEM((NCHUNK, CH), jnp.int32),
          pltpu.VMEM((4, CH), jnp.float32),
          pltpu.SemaphoreType.DMA,
          pltpu.SemaphoreType.DMA,
          pltpu.SemaphoreType.DMA,
          pltpu.SemaphoreType.DMA,
      ],
      compiler_params=pltpu.CompilerParams(use_tc_tiling_on_sc=False),
  )
  return fn(q, src, dst)


_GRID1 = 10
_BLK1 = N // _GRID1


def _pre_body(x_ref, wl_ref, wr_ref, b_ref, p_ref, r_ref):
  xb = x_ref[...]
  p_ref[...] = jnp.dot(xb, wl_ref[...], preferred_element_type=jnp.float32)
  r_ref[...] = jnp.dot(xb, wr_ref[...],
                       preferred_element_type=jnp.float32) + b_ref[...]


def _pre_call(x, Wl, Wr, b):
  return pl.pallas_call(
      _pre_body,
      grid=(_GRID1,),
      in_specs=[
          pl.BlockSpec((_BLK1, D), lambda i: (i, _Z)),
          pl.BlockSpec((D, H), lambda i: (_Z, _Z)),
          pl.BlockSpec((D, H), lambda i: (_Z, _Z)),
          pl.BlockSpec((1, H), lambda i: (_Z, _Z)),
      ],
      out_specs=[
          pl.BlockSpec((_BLK1, H), lambda i: (i, _Z)),
          pl.BlockSpec((_BLK1, H), lambda i: (i, _Z)),
      ],
      out_shape=[
          jax.ShapeDtypeStruct((NP, H), jnp.float32),
          jax.ShapeDtypeStruct((N, H), jnp.float32),
      ],
  )(x, Wl, Wr, b)


def _mid_body(has_res, with_counts, *refs):
  it = iter(refs)
  s_ref = next(it); cnt_ref = next(it); r_ref = next(it)
  res_ref = next(it) if has_res else None
  bat_ref = next(it) if with_counts else None
  wl_ref = next(it); wr_ref = next(it); b_ref = next(it)
  x_ref = next(it); p_ref = next(it); rn_ref = next(it)
  counts_ref = next(it) if with_counts else None

  cnt = cnt_ref[0] + cnt_ref[1]
  m = (s_ref[0] + s_ref[1]) / jnp.maximum(cnt, 1.0)
  x_new = jnp.maximum(m + r_ref[...], 0.0)
  if has_res:
    x_new = x_new + res_ref[...]
  x_ref[...] = x_new
  p_ref[...] = jnp.dot(x_new, wl_ref[...], preferred_element_type=jnp.float32)
  rn_ref[...] = jnp.dot(x_new, wr_ref[...],
                        preferred_element_type=jnp.float32) + b_ref[...]
  if with_counts:
    gids = lax.broadcasted_iota(jnp.int32, (_BLK1, B), 1)
    oh = (bat_ref[...] == gids).astype(jnp.float32)
    counts_ref[...] = jnp.sum(oh, axis=0, keepdims=True)[None]


def _mid_call(s_pair, cnt_pair, r, res, bat, Wl, Wr, b):
  has_res = res is not None
  with_counts = bat is not None
  hn = Wl.shape[1]
  in_specs = [
      pl.BlockSpec((NSC, _BLK1, H), lambda i: (_Z, i, _Z)),
      pl.BlockSpec((NSC, _BLK1, 1), lambda i: (_Z, i, _Z)),
      pl.BlockSpec((_BLK1, H), lambda i: (i, _Z)),
  ]
  args = [s_pair, cnt_pair, r]
  if has_res:
    in_specs.append(pl.BlockSpec((_BLK1, H), lambda i: (i, _Z)))
    args.append(res)
  if with_counts:
    in_specs.append(pl.BlockSpec((_BLK1, 1), lambda i: (i, _Z)))
    args.append(bat)
  in_specs += [
      pl.BlockSpec((H, hn), lambda i: (_Z, _Z)),
      pl.BlockSpec((H, hn), lambda i: (_Z, _Z)),
      pl.BlockSpec((1, hn), lambda i: (_Z, _Z)),
  ]
  args += [Wl, Wr, b]
  out_specs = [
      pl.BlockSpec((_BLK1, H), lambda i: (i, _Z)),
      pl.BlockSpec((_BLK1, hn), lambda i: (i, _Z)),
      pl.BlockSpec((_BLK1, hn), lambda i: (i, _Z)),
  ]
  out_shape = [
      jax.ShapeDtypeStruct((N, H), jnp.float32),
      jax.ShapeDtypeStruct((NP, hn), jnp.float32),
      jax.ShapeDtypeStruct((N, hn), jnp.float32),
  ]
  if with_counts:
    out_specs.append(pl.BlockSpec((1, 1, B), lambda i: (i, _Z, _Z)))
    out_shape.append(jax.ShapeDtypeStruct((_GRID1, 1, B), jnp.float32))
  return pl.pallas_call(
      functools.partial(_mid_body, has_res, with_counts),
      grid=(_GRID1,),
      in_specs=in_specs,
      out_specs=out_specs,
      out_shape=out_shape,
  )(*args)


_GRIDF = 25
_BLKF = N // _GRIDF
_JW = 1024
_NJ = NP // _JW


def _final_body(x3_ref, saggc_ref, qpc_ref, batc_ref, saggr_ref, qpr_ref,
                batr_ref, counts_ref, wlin_ref, blin_ref, out_ref,
                sums_ref, cntk_ref):
  pid = pl.program_id(0)

  score_c = saggc_ref[0] + saggc_ref[1] + qpc_ref[...]
  b_c = batc_ref[...]
  i_c = lax.broadcasted_iota(jnp.int32, (_BLKF, 1), 0) + pid * _BLKF

  bimin = jnp.min(b_c)
  bimax = jnp.max(b_c)

  def jstep(jt, acc):
    sr = saggr_ref[0, jt] + saggr_ref[1, jt] + qpr_ref[jt]
    br = batr_ref[jt]
    ir = lax.broadcasted_iota(jnp.int32, (1, _JW), 1) + jt * _JW
    overlap = (jnp.min(br) <= bimax) & (jnp.max(br) >= bimin)

    def hit(a):
      gt = (sr > score_c) | ((sr == score_c) & (ir < i_c))
      same = br == b_c
      return a + jnp.sum((gt & same).astype(jnp.float32), axis=1,
                         keepdims=True)
    return lax.cond(overlap, hit, lambda a: a, acc)

  rank = lax.fori_loop(jnp.int32(0), jnp.int32(_NJ), jstep,
                       jnp.zeros((_BLKF, 1), jnp.float32))

  counts = jnp.sum(counts_ref[...], axis=0)
  k_per = jnp.floor((counts + 1.0) * 0.5)
  gids = lax.broadcasted_iota(jnp.int32, (_BLKF, B), 1)
  oh = (b_c == gids).astype(jnp.float32)
  k_node = jnp.sum(oh * k_per, axis=1, keepdims=True)

  keep = (rank < k_node).astype(jnp.float32)
  gate = jnp.tanh(score_c) * keep
  gated = x3_ref[...] * gate

  part_sums = lax.dot_general(oh, gated, (((0,), (0,)), ((), ())),
                              preferred_element_type=jnp.float32)
  part_cnt = jnp.sum(oh * keep, axis=0, keepdims=True)

  @pl.when(pid == 0)
  def _init():
    sums_ref[...] = part_sums
    cntk_ref[...] = part_cnt

  @pl.when(pid > 0)
  def _acc():
    sums_ref[...] += part_sums
    cntk_ref[...] += part_cnt

  @pl.when(pid == _GRIDF - 1)
  def _fin():
    denom = jnp.maximum(cntk_ref[...], 1.0)
    pooled = sums_ref[...] / denom.reshape(B, 1)
    logits = jnp.dot(pooled, wlin_ref[...],
                     preferred_element_type=jnp.float32) + blin_ref[...]
    mx = jnp.max(logits, axis=1, keepdims=True)
    lse = mx + jnp.log(jnp.sum(jnp.exp(logits - mx), axis=1, keepdims=True))
    out_ref[...] = logits - lse


def _final_call(x3, sagg_col, qpob_col, bat_col, sagg_row, qpob_row, bat_row,
                counts, Wlin, blin):
  return pl.pallas_call(
      _final_body,
      grid=(_GRIDF,),
      in_specs=[
          pl.BlockSpec((_BLKF, H), lambda i: (i, _Z)),
          pl.BlockSpec((NSC, _BLKF, 1), lambda i: (_Z, i, _Z)),
          pl.BlockSpec((_BLKF, 1), lambda i: (i, _Z)),
          pl.BlockSpec((_BLKF, 1), lambda i: (i, _Z)),
          pl.BlockSpec((NSC, _NJ, 1, _JW), lambda i: (_Z, _Z, _Z, _Z)),
          pl.BlockSpec((_NJ, 1, _JW), lambda i: (_Z, _Z, _Z)),
          pl.BlockSpec((_NJ, 1, _JW), lambda i: (_Z, _Z, _Z)),
          pl.BlockSpec((_GRID1, 1, B), lambda i: (_Z, _Z, _Z)),
          pl.BlockSpec((H, C), lambda i: (_Z, _Z)),
          pl.BlockSpec((1, C), lambda i: (_Z, _Z)),
      ],
      out_specs=pl.BlockSpec((B, C), lambda i: (_Z, _Z)),
      out_shape=jax.ShapeDtypeStruct((B, C), jnp.float32),
      scratch_shapes=[
          pltpu.VMEM((B, H), jnp.float32),
          pltpu.VMEM((1, B), jnp.float32),
      ],
  )(x3, sagg_col, qpob_col, bat_col, sagg_row, qpob_row, bat_row, counts,
    Wlin, blin)


def kernel(x, edge_index, batch, W1l, b1, W1r, W2l, b2, W2r, W3l, b3, W3r,
           Wpr, bpr, Wpo, Wlin, blin):
  x = x.astype(jnp.float32)
  src = edge_index[0].astype(jnp.int32).reshape(NSC * NTILE, NCHUNK, CH)
  dst = edge_index[1].astype(jnp.int32).reshape(NSC * NTILE, NCHUNK, CH)
  bat = batch.astype(jnp.int32)
  bat_col = bat.reshape(N, 1)
  bat_row = jnp.pad(bat, (0, NP - N), constant_values=-1).reshape(_NJ, 1, _JW)

  p1, r1 = _pre_call(x, W1l, W1r, b1.reshape(1, H))
  s1_flat, cnt_flat = _seg_call(p1, src, dst, with_cnt=True)
  s1 = s1_flat.reshape(NSC, NP, H)
  cnt = cnt_flat.reshape(NSC, NP, 1)

  x1, p2, r2 = _mid_call(s1, cnt, r1, None, None, W2l, W2r, b2.reshape(1, H))
  s2 = _seg_call(p2, src, dst, with_cnt=False)[0].reshape(NSC, NP, H)

  x2, p3, r3 = _mid_call(s2, cnt, r2, x1, None, W3l, W3r, b3.reshape(1, H))
  s3 = _seg_call(p3, src, dst, with_cnt=False)[0].reshape(NSC, NP, H)

  x3, qpr, qpob, counts = _mid_call(
      s3, cnt, r3, x2, bat_col, Wpr, Wpo, bpr.reshape(1, 1))

  sagg_flat = _score_call(qpr.reshape(NP), src, dst)
  sagg_col = sagg_flat.reshape(NSC, NP, 1)
  sagg_row = sagg_flat.reshape(NSC, _NJ, 1, _JW)
  qpob_row = jnp.pad(qpob.reshape(N), (0, NP - N)).reshape(_NJ, 1, _JW)

  return _final_call(x3, sagg_col, qpob.reshape(N, 1), bat_col, sagg_row,
                     qpob_row, bat_row, counts, Wlin, blin.reshape(1, C))


# speedup vs baseline: 1.1065x; 1.0127x over previous
import functools

import jax
import jax.numpy as jnp
import numpy as np
from jax import lax
from jax.experimental import pallas as pl
from jax.experimental.pallas import tpu as pltpu
from jax.experimental.pallas import tpu_sc as plsc

N = 10000
E = 320000
D = 128
H = 64
C = 10
B = 64

NP = 10240
NSC = 2
NTILE = 16
TPB = NP // NTILE
CH = 80
EPW = E // (NSC * NTILE)
NCHUNK = EPW // CH

_Z = np.int32(0)

_mesh = plsc.VectorSubcoreMesh(
    core_axis_name="c", subcore_axis_name="s", num_cores=NSC,
    num_subcores=NTILE)


def _seg_body(with_cnt, p_hbm, src_hbm, dst_hbm, s_out, *rest):
  if with_cnt:
    (cnt_out, s_sh, cnt_sh, p_sh, zbuf, zbuf1, ones_v, sidx, didx, rows,
     gsem0, gsem1, gsem2, gsem3) = rest
  else:
    (s_sh, p_sh, zbuf, zbuf1, sidx, didx, rows, gsem0, gsem1, gsem2,
     gsem3) = rest
    cnt_out = cnt_sh = ones_v = None
  c = lax.axis_index("c")
  s = lax.axis_index("s")
  w = c * NTILE + s

  pltpu.sync_copy(p_hbm.at[pl.ds(s * TPB, TPB)], p_sh.at[pl.ds(s * TPB, TPB)])
  pltpu.sync_copy(src_hbm.at[w], sidx)
  pltpu.sync_copy(dst_hbm.at[w], didx)
  def zb(r, carry):
    for q in range(H // 16):
      zbuf[r, pl.ds(q * 16, 16)] = jnp.zeros((16,), jnp.float32)
    return carry
  lax.fori_loop(jnp.int32(0), jnp.int32(64), zb, jnp.int32(0))
  for d in range(TPB // 64):
    pltpu.sync_copy(zbuf, s_sh.at[pl.ds(s * TPB + d * 64, 64)])
  if with_cnt:
    def zb1(i, carry):
      zbuf1[pl.ds(i * 16, 16)] = jnp.zeros((16,), jnp.float32)
      return carry
    lax.fori_loop(jnp.int32(0), jnp.int32(TPB // 16), zb1, jnp.int32(0))
    for q in range(CH // 16):
      ones_v[pl.ds(q * 16, 16)] = jnp.ones((16,), jnp.float32)
    pltpu.sync_copy(zbuf1, cnt_sh.at[pl.ds(s * TPB, TPB)])
  plsc.subcore_barrier()

  sems = (gsem0, gsem1, gsem2, gsem3)

  def issue(k, b):
    pltpu.async_copy(p_sh.at[sidx.at[k]], rows.at[np.int32(b)], sems[b])

  def step(k, b):
    pltpu.make_async_copy(p_sh.at[sidx.at[_Z]], rows.at[np.int32(b)],
                          sems[b]).wait()
    pltpu.sync_copy(rows.at[np.int32(b)], s_sh.at[didx.at[k]], add=True)
    if with_cnt:
      pltpu.sync_copy(ones_v, cnt_sh.at[didx.at[k]], add=True)

  for b in range(4):
    issue(jnp.int32(b), b)

  def quad(p, carry):
    k0 = p * 4
    for b in range(4):
      step(k0 + b, b)
      issue(k0 + b + 4, b)
    return carry
  nquad = (NCHUNK - 5) // 4
  lax.fori_loop(jnp.int32(0), jnp.int32(nquad), quad, jnp.int32(0))
  step(jnp.int32(NCHUNK - 5), 0)
  issue(jnp.int32(NCHUNK - 1), 0)
  step(jnp.int32(NCHUNK - 4), 1)
  step(jnp.int32(NCHUNK - 3), 2)
  step(jnp.int32(NCHUNK - 2), 3)
  step(jnp.int32(NCHUNK - 1), 0)
  plsc.subcore_barrier()

  row0 = c * NP + s * TPB
  pltpu.sync_copy(s_sh.at[pl.ds(s * TPB, TPB)], s_out.at[pl.ds(row0, TPB)])
  if with_cnt:
    pltpu.sync_copy(cnt_sh.at[pl.ds(s * TPB, TPB)],
                    cnt_out.at[pl.ds(row0, TPB)])


def _seg_call(p, src, dst, with_cnt):
  out_type = [jax.ShapeDtypeStruct((NSC * NP, H), jnp.float32)]
  scratch = []
  if with_cnt:
    out_type.append(jax.ShapeDtypeStruct((NSC * NP,), jnp.float32))
  scratch.append(pltpu.VMEM_SHARED((NP, H), jnp.float32))
  if with_cnt:
    scratch.append(pltpu.VMEM_SHARED((NP,), jnp.float32))
  scratch.append(pltpu.VMEM_SHARED((NP, H), jnp.float32))
  scratch += [
      pltpu.VMEM((64, H), jnp.float32),
      pltpu.VMEM((TPB,), jnp.float32),
  ]
  if with_cnt:
    scratch.append(pltpu.VMEM((CH,), jnp.float32))
  scratch += [
      pltpu.VMEM((NCHUNK, CH), jnp.int32),
      pltpu.VMEM((NCHUNK, CH), jnp.int32),
      pltpu.VMEM((4, CH, H), jnp.float32),
      pltpu.SemaphoreType.DMA,
      pltpu.SemaphoreType.DMA,
      pltpu.SemaphoreType.DMA,
      pltpu.SemaphoreType.DMA,
  ]
  fn = pl.kernel(
      functools.partial(_seg_body, with_cnt),
      out_type=out_type,
      mesh=_mesh,
      scratch_types=scratch,
      compiler_params=pltpu.CompilerParams(use_tc_tiling_on_sc=False),
  )
  return fn(p, src, dst)


def _score_body(q_hbm, src_hbm, dst_hbm, g_out, g_sh, q_sh, zbuf1, sidx,
                didx, vals, gsem0, gsem1, gsem2, gsem3):
  c = lax.axis_index("c")
  s = lax.axis_index("s")
  w = c * NTILE + s

  pltpu.sync_copy(q_hbm.at[pl.ds(s * TPB, TPB)], q_sh.at[pl.ds(s * TPB, TPB)])
  pltpu.sync_copy(src_hbm.at[w], sidx)
  pltpu.sync_copy(dst_hbm.at[w], didx)
  def zb1(i, carry):
    zbuf1[pl.ds(i * 16, 16)] = jnp.zeros((16,), jnp.float32)
    return carry
  lax.fori_loop(jnp.int32(0), jnp.int32(TPB // 16), zb1, jnp.int32(0))
  pltpu.sync_copy(zbuf1, g_sh.at[pl.ds(s * TPB, TPB)])
  plsc.subcore_barrier()

  sems = (gsem0, gsem1, gsem2, gsem3)

  def issue(k, b):
    pltpu.async_copy(q_sh.at[sidx.at[k]], vals.at[np.int32(b)], sems[b])

  def step(k, b):
    pltpu.make_async_copy(q_sh.at[sidx.at[_Z]], vals.at[np.int32(b)],
                          sems[b]).wait()
    pltpu.sync_copy(vals.at[np.int32(b)], g_sh.at[didx.at[k]], add=True)

  for b in range(4):
    issue(jnp.int32(b), b)

  def quad(p, carry):
    k0 = p * 4
    for b in range(4):
      step(k0 + b, b)
      issue(k0 + b + 4, b)
    return carry
  nquad = (NCHUNK - 5) // 4
  lax.fori_loop(jnp.int32(0), jnp.int32(nquad), quad, jnp.int32(0))
  step(jnp.int32(NCHUNK - 5), 0)
  issue(jnp.int32(NCHUNK - 1), 0)
  step(jnp.int32(NCHUNK - 4), 1)
  step(jnp.int32(NCHUNK - 3), 2)
  step(jnp.int32(NCHUNK - 2), 3)
  step(jnp.int32(NCHUNK - 1), 0)
  plsc.subcore_barrier()

  row0 = c * NP + s * TPB
  pltpu.sync_copy(g_sh.at[pl.ds(s * TPB, TPB)], g_out.at[pl.ds(row0, TPB)])


def _score_call(q, src, dst):
  fn = pl.kernel(
      _score_body,
      out_type=jax.ShapeDtypeStruct((NSC * NP,), jnp.float32),
      mesh=_mesh,
      scratch_types=[
          pltpu.VMEM_SHARED((NP,), jnp.float32),
          pltpu.VMEM_SHARED((NP,), jnp.float32),
          pltpu.VMEM((TPB,), jnp.float32),
          pltpu.VMEM((NCHUNK, CH), jnp.int32),
          pltpu.VMEM((NCHUNK, CH), jnp.int32),
          pltpu.VMEM((4, CH), jnp.float32),
          pltpu.SemaphoreType.DMA,
          pltpu.SemaphoreType.DMA,
          pltpu.SemaphoreType.DMA,
          pltpu.SemaphoreType.DMA,
      ],
      compiler_params=pltpu.CompilerParams(use_tc_tiling_on_sc=False),
  )
  return fn(q, src, dst)


_GRID1 = 10
_BLK1 = N // _GRID1


def _pre_body(x_ref, wl_ref, wr_ref, b_ref, p_ref, r_ref):
  xb = x_ref[...]
  p_ref[...] = jnp.dot(xb, wl_ref[...], preferred_element_type=jnp.float32)
  r_ref[...] = jnp.dot(xb, wr_ref[...],
                       preferred_element_type=jnp.float32) + b_ref[...]


def _pre_call(x, Wl, Wr, b):
  return pl.pallas_call(
      _pre_body,
      grid=(_GRID1,),
      in_specs=[
          pl.BlockSpec((_BLK1, D), lambda i: (i, _Z)),
          pl.BlockSpec((D, H), lambda i: (_Z, _Z)),
          pl.BlockSpec((D, H), lambda i: (_Z, _Z)),
          pl.BlockSpec((1, H), lambda i: (_Z, _Z)),
      ],
      out_specs=[
          pl.BlockSpec((_BLK1, H), lambda i: (i, _Z)),
          pl.BlockSpec((_BLK1, H), lambda i: (i, _Z)),
      ],
      out_shape=[
          jax.ShapeDtypeStruct((NP, H), jnp.float32),
          jax.ShapeDtypeStruct((N, H), jnp.float32),
      ],
  )(x, Wl, Wr, b)


def _mid_body(has_res, with_counts, *refs):
  it = iter(refs)
  s_ref = next(it); cnt_ref = next(it); r_ref = next(it)
  res_ref = next(it) if has_res else None
  bat_ref = next(it) if with_counts else None
  wl_ref = next(it); wr_ref = next(it); b_ref = next(it)
  x_ref = next(it); p_ref = next(it); rn_ref = next(it)
  counts_ref = next(it) if with_counts else None

  cnt = cnt_ref[0] + cnt_ref[1]
  m = (s_ref[0] + s_ref[1]) / jnp.maximum(cnt, 1.0)
  x_new = jnp.maximum(m + r_ref[...], 0.0)
  if has_res:
    x_new = x_new + res_ref[...]
  x_ref[...] = x_new
  p_ref[...] = jnp.dot(x_new, wl_ref[...], preferred_element_type=jnp.float32)
  rn_ref[...] = jnp.dot(x_new, wr_ref[...],
                        preferred_element_type=jnp.float32) + b_ref[...]
  if with_counts:
    gids = lax.broadcasted_iota(jnp.int32, (_BLK1, B), 1)
    oh = (bat_ref[...] == gids).astype(jnp.float32)
    counts_ref[...] = jnp.sum(oh, axis=0, keepdims=True)[None]


def _mid_call(s_pair, cnt_pair, r, res, bat, Wl, Wr, b):
  has_res = res is not None
  with_counts = bat is not None
  hn = Wl.shape[1]
  in_specs = [
      pl.BlockSpec((NSC, _BLK1, H), lambda i: (_Z, i, _Z)),
      pl.BlockSpec((NSC, _BLK1, 1), lambda i: (_Z, i, _Z)),
      pl.BlockSpec((_BLK1, H), lambda i: (i, _Z)),
  ]
  args = [s_pair, cnt_pair, r]
  if has_res:
    in_specs.append(pl.BlockSpec((_BLK1, H), lambda i: (i, _Z)))
    args.append(res)
  if with_counts:
    in_specs.append(pl.BlockSpec((_BLK1, 1), lambda i: (i, _Z)))
    args.append(bat)
  in_specs += [
      pl.BlockSpec((H, hn), lambda i: (_Z, _Z)),
      pl.BlockSpec((H, hn), lambda i: (_Z, _Z)),
      pl.BlockSpec((1, hn), lambda i: (_Z, _Z)),
  ]
  args += [Wl, Wr, b]
  out_specs = [
      pl.BlockSpec((_BLK1, H), lambda i: (i, _Z)),
      pl.BlockSpec((_BLK1, hn), lambda i: (i, _Z)),
      pl.BlockSpec((_BLK1, hn), lambda i: (i, _Z)),
  ]
  out_shape = [
      jax.ShapeDtypeStruct((N, H), jnp.float32),
      jax.ShapeDtypeStruct((NP, hn), jnp.float32),
      jax.ShapeDtypeStruct((N, hn), jnp.float32),
  ]
  if with_counts:
    out_specs.append(pl.BlockSpec((1, 1, B), lambda i: (i, _Z, _Z)))
    out_shape.append(jax.ShapeDtypeStruct((_GRID1, 1, B), jnp.float32))
  return pl.pallas_call(
      functools.partial(_mid_body, has_res, with_counts),
      grid=(_GRID1,),
      in_specs=in_specs,
      out_specs=out_specs,
      out_shape=out_shape,
  )(*args)


_GRIDF = 25
_BLKF = N // _GRIDF
_JW = 1024
_NJ = NP // _JW


def _final_body(x3_ref, saggc_ref, qpc_ref, batc_ref, saggr_ref, qpr_ref,
                batr_ref, counts_ref, wlin_ref, blin_ref, out_ref,
                sums_ref, cntk_ref):
  pid = pl.program_id(0)

  score_c = saggc_ref[0] + saggc_ref[1] + qpc_ref[...]
  b_c = batc_ref[...]
  i_c = lax.broadcasted_iota(jnp.int32, (_BLKF, 1), 0) + pid * _BLKF

  bimin = jnp.min(b_c)
  bimax = jnp.max(b_c)

  def keys(score, b, idx):
    si = lax.bitcast_convert_type(score, jnp.int32)
    u = lax.bitcast_convert_type(
        si ^ ((si >> 31) | np.int32(-2147483648)), jnp.uint32)
    bu = b.astype(jnp.uint32)
    k1 = (bu << 25) | (u >> 7)
    k2 = ((u & np.uint32(127)) << 14) | (np.uint32(16383) -
                                         idx.astype(jnp.uint32))
    return k1, k2

  k1c, k2c = keys(score_c, b_c, i_c)

  def jstep(jt, acc):
    sr = saggr_ref[0, jt] + saggr_ref[1, jt] + qpr_ref[jt]
    br = batr_ref[jt]
    ir = lax.broadcasted_iota(jnp.int32, (1, _JW), 1) + jt * _JW
    bjmin = jnp.min(br)
    bjmax = jnp.max(br)
    overlap = (bjmin <= bimax) & (bjmax >= bimin)

    def hit(a):
      k1r, k2r = keys(sr, br, ir)
      k1r = jnp.where(br < 0, np.uint32(0), k1r)
      better = (k1r > k1c) | ((k1r == k1c) & (k2r > k2c))
      return a + jnp.sum(better.astype(jnp.float32), axis=1, keepdims=True)

    def skp(a):
      nreal = jnp.clip(np.float32(N) - jt.astype(jnp.float32) * _JW,
                       0.0, np.float32(_JW))
      return a + jnp.where(bjmin > bimax, nreal, 0.0)
    return lax.cond(overlap, hit, skp, acc)

  total = lax.fori_loop(jnp.int32(0), jnp.int32(_NJ), jstep,
                        jnp.zeros((_BLKF, 1), jnp.float32))

  counts = jnp.sum(counts_ref[...], axis=0)
  k_per = jnp.floor((counts + 1.0) * 0.5)
  gids = lax.broadcasted_iota(jnp.int32, (_BLKF, B), 1)
  oh = (b_c == gids).astype(jnp.float32)
  k_node = jnp.sum(oh * k_per, axis=1, keepdims=True)
  ltri = (lax.broadcasted_iota(jnp.int32, (B, B), 0) <=
          lax.broadcasted_iota(jnp.int32, (B, B), 1)).astype(jnp.float32)
  ends = jnp.dot(counts, ltri, preferred_element_type=jnp.float32)
  end_node = jnp.sum(oh * ends, axis=1, keepdims=True)
  rank = total - (float(N) - end_node)

  keep = (rank < k_node).astype(jnp.float32)
  gate = jnp.tanh(score_c) * keep
  gated = x3_ref[...] * gate

  part_sums = lax.dot_general(oh, gated, (((0,), (0,)), ((), ())),
                              preferred_element_type=jnp.float32)
  part_cnt = jnp.sum(oh * keep, axis=0, keepdims=True)

  @pl.when(pid == 0)
  def _init():
    sums_ref[...] = part_sums
    cntk_ref[...] = part_cnt

  @pl.when(pid > 0)
  def _acc():
    sums_ref[...] += part_sums
    cntk_ref[...] += part_cnt

  @pl.when(pid == _GRIDF - 1)
  def _fin():
    denom = jnp.maximum(cntk_ref[...], 1.0)
    pooled = sums_ref[...] / denom.reshape(B, 1)
    logits = jnp.dot(pooled, wlin_ref[...],
                     preferred_element_type=jnp.float32) + blin_ref[...]
    mx = jnp.max(logits, axis=1, keepdims=True)
    lse = mx + jnp.log(jnp.sum(jnp.exp(logits - mx), axis=1, keepdims=True))
    out_ref[...] = logits - lse


def _final_call(x3, sagg_col, qpob_col, bat_col, sagg_row, qpob_row, bat_row,
                counts, Wlin, blin):
  return pl.pallas_call(
      _final_body,
      grid=(_GRIDF,),
      in_specs=[
          pl.BlockSpec((_BLKF, H), lambda i: (i, _Z)),
          pl.BlockSpec((NSC, _BLKF, 1), lambda i: (_Z, i, _Z)),
          pl.BlockSpec((_BLKF, 1), lambda i: (i, _Z)),
          pl.BlockSpec((_BLKF, 1), lambda i: (i, _Z)),
          pl.BlockSpec((NSC, _NJ, 1, _JW), lambda i: (_Z, _Z, _Z, _Z)),
          pl.BlockSpec((_NJ, 1, _JW), lambda i: (_Z, _Z, _Z)),
          pl.BlockSpec((_NJ, 1, _JW), lambda i: (_Z, _Z, _Z)),
          pl.BlockSpec((_GRID1, 1, B), lambda i: (_Z, _Z, _Z)),
          pl.BlockSpec((H, C), lambda i: (_Z, _Z)),
          pl.BlockSpec((1, C), lambda i: (_Z, _Z)),
      ],
      out_specs=pl.BlockSpec((B, C), lambda i: (_Z, _Z)),
      out_shape=jax.ShapeDtypeStruct((B, C), jnp.float32),
      scratch_shapes=[
          pltpu.VMEM((B, H), jnp.float32),
          pltpu.VMEM((1, B), jnp.float32),
      ],
  )(x3, sagg_col, qpob_col, bat_col, sagg_row, qpob_row, bat_row, counts,
    Wlin, blin)


def kernel(x, edge_index, batch, W1l, b1, W1r, W2l, b2, W2r, W3l, b3, W3r,
           Wpr, bpr, Wpo, Wlin, blin):
  x = x.astype(jnp.float32)
  src = edge_index[0].astype(jnp.int32).reshape(NSC * NTILE, NCHUNK, CH)
  dst = edge_index[1].astype(jnp.int32).reshape(NSC * NTILE, NCHUNK, CH)
  bat = batch.astype(jnp.int32)
  bat_col = bat.reshape(N, 1)
  bat_row = jnp.pad(bat, (0, NP - N), constant_values=-1).reshape(_NJ, 1, _JW)

  p1, r1 = _pre_call(x, W1l, W1r, b1.reshape(1, H))
  s1_flat, cnt_flat = _seg_call(p1, src, dst, with_cnt=True)
  s1 = s1_flat.reshape(NSC, NP, H)
  cnt = cnt_flat.reshape(NSC, NP, 1)

  x1, p2, r2 = _mid_call(s1, cnt, r1, None, None, W2l, W2r, b2.reshape(1, H))
  s2 = _seg_call(p2, src, dst, with_cnt=False)[0].reshape(NSC, NP, H)

  x2, p3, r3 = _mid_call(s2, cnt, r2, x1, None, W3l, W3r, b3.reshape(1, H))
  s3 = _seg_call(p3, src, dst, with_cnt=False)[0].reshape(NSC, NP, H)

  x3, qpr, qpob, counts = _mid_call(
      s3, cnt, r3, x2, bat_col, Wpr, Wpo, bpr.reshape(1, 1))

  sagg_flat = _score_call(qpr.reshape(NP), src, dst)
  sagg_col = sagg_flat.reshape(NSC, NP, 1)
  sagg_row = sagg_flat.reshape(NSC, _NJ, 1, _JW)
  qpob_row = jnp.pad(qpob.reshape(N), (0, NP - N)).reshape(_NJ, 1, _JW)

  return _final_call(x3, sagg_col, qpob.reshape(N, 1), bat_col, sagg_row,
                     qpob_row, bat_row, counts, Wlin, blin.reshape(1, C))
